# baseline (speedup 1.0000x reference)
import functools

import jax
import jax.numpy as jnp
from jax import lax
from jax.experimental import pallas as pl
from jax.experimental.pallas import tpu as pltpu
from jax.experimental.pallas import tpu_sc as plsc

N_REC = 3
D = 768
NW = 32
TILE = 64
D_UNROLL = 1


def _sc_router(xf, W):
    N = xf.shape[0]
    tps = N // NW
    nt = tps // TILE
    mesh = plsc.VectorSubcoreMesh(core_axis_name="c", subcore_axis_name="s")

    @functools.partial(
        pl.kernel,
        out_type=[
            jax.ShapeDtypeStruct((N * N_REC,), jnp.float32),
            jax.ShapeDtypeStruct((N,), jnp.float32),
        ],
        mesh=mesh,
        scratch_types=[
            pltpu.VMEM((N_REC, D), jnp.float32),
            pltpu.VMEM((TILE * D,), jnp.float32),
            pltpu.VMEM((TILE * D,), jnp.float32),
            pltpu.VMEM((tps * N_REC,), jnp.float32),
            pltpu.VMEM((tps,), jnp.float32),
            pltpu.SemaphoreType.DMA,
            pltpu.SemaphoreType.DMA,
        ],
        compiler_params=pltpu.CompilerParams(
            use_tc_tiling_on_sc=False, needs_layout_passes=False),
    )
    def k(x_hbm, w_hbm, lg_hbm, gt_hbm, w_v, xb0, xb1, lg_v, gt_v, sem0, sem1):
        wid = lax.axis_index("s") * 2 + lax.axis_index("c")
        base = wid * tps
        pltpu.sync_copy(w_hbm, w_v)
        iota = lax.broadcasted_iota(jnp.int32, (16,), 0)

        def start_copy(t, buf, sem):
            pltpu.async_copy(
                x_hbm.at[pl.ds((base + t * TILE) * D, TILE * D)], buf, sem)

        def wait_copy(buf, sem):
            pltpu.make_async_copy(
                x_hbm.at[pl.ds(base * D, TILE * D)], buf, sem).wait()

        def compute_tile(t, buf):
            for g in range(TILE // 16):
                idx0 = (g * 16 + iota) * D

                @pl.loop(0, D // 16,
                         init_carry=(jnp.zeros((16,), jnp.float32),) * 3,
                         unroll=D_UNROLL)
                def accs(c, carry):
                    a0, a1, a2 = carry
                    w0 = w_v[0, pl.ds(c * 16, 16)]
                    w1 = w_v[1, pl.ds(c * 16, 16)]
                    w2 = w_v[2, pl.ds(c * 16, 16)]
                    idxc = idx0 + c * 16
                    for j in range(16):
                        xv = plsc.load_gather(buf, [idxc + j])
                        a0 = a0 + xv * w0[j]
                        a1 = a1 + xv * w1[j]
                        a2 = a2 + xv * w2[j]
                    return (a0, a1, a2)

                l0, l1, l2 = accs
                m = jnp.maximum(jnp.maximum(l0, l1), l2)
                ssum = jnp.exp(l0 - m) + jnp.exp(l1 - m) + jnp.exp(l2 - m)
                off = t * TILE + g * 16
                gt_v[pl.ds(off, 16)] = 1.0 / ssum
                oidx = (off + iota) * N_REC
                plsc.store_scatter(lg_v, [oidx], l0)
                plsc.store_scatter(lg_v, [oidx + 1], l1)
                plsc.store_scatter(lg_v, [oidx + 2], l2)

        start_copy(0, xb0, sem0)
        start_copy(1, xb1, sem1)

        @pl.loop(0, nt, step=2)
        def _(t):
            wait_copy(xb0, sem0)
            compute_tile(t, xb0)

            @pl.when(t + 2 < nt)
            def _():
                start_copy(t + 2, xb0, sem0)

            wait_copy(xb1, sem1)
            compute_tile(t + 1, xb1)

            @pl.when(t + 3 < nt)
            def _():
                start_copy(t + 3, xb1, sem1)

        pltpu.sync_copy(lg_v, lg_hbm.at[pl.ds(base * N_REC, tps * N_REC)])
        pltpu.sync_copy(gt_v, gt_hbm.at[pl.ds(base, tps)])

    return k(xf.reshape(N * D), W)


def kernel(x, W):
    B, T, _ = x.shape
    N = B * T
    xf = x.reshape(N, D)
    lg, gt = _sc_router(xf, W)
    selected = jnp.broadcast_to(
        jnp.arange(T, dtype=jnp.int32)[None, :, None], (B, T, 1))
    return selected, gt.reshape(B, T, 1), lg.reshape(B, T, N_REC)


# speedup vs baseline: 2.5419x; 2.5419x over previous
import functools

import jax
import jax.numpy as jnp
from jax import lax
from jax.experimental import pallas as pl
from jax.experimental.pallas import tpu as pltpu
from jax.experimental.pallas import tpu_sc as plsc

N_REC = 3
D = 768
NW = 32
TILE = 64
D_UNROLL = 2
TB = 8


def _sc_router(xf, W):
    N = xf.shape[0]
    tps = N // NW
    nt = tps // TILE
    mesh = plsc.VectorSubcoreMesh(core_axis_name="c", subcore_axis_name="s")

    @functools.partial(
        pl.kernel,
        out_type=[
            jax.ShapeDtypeStruct((N * N_REC,), jnp.float32),
            jax.ShapeDtypeStruct((N,), jnp.float32),
        ],
        mesh=mesh,
        scratch_types=[
            pltpu.VMEM((N_REC, D), jnp.float32),
            pltpu.VMEM((TILE * D,), jnp.float32),
            pltpu.VMEM((TILE * D,), jnp.float32),
            pltpu.VMEM((tps * N_REC,), jnp.float32),
            pltpu.VMEM((tps,), jnp.float32),
            pltpu.SemaphoreType.DMA,
            pltpu.SemaphoreType.DMA,
        ],
        compiler_params=pltpu.CompilerParams(
            use_tc_tiling_on_sc=False, needs_layout_passes=False),
    )
    def k(x_hbm, w_hbm, lg_hbm, gt_hbm, w_v, xb0, xb1, lg_v, gt_v, sem0, sem1):
        wid = lax.axis_index("s") * 2 + lax.axis_index("c")
        base = wid * tps
        pltpu.sync_copy(w_hbm, w_v)
        iota = lax.broadcasted_iota(jnp.int32, (16,), 0)

        def start_copy(t, buf, sem):
            pltpu.async_copy(
                x_hbm.at[pl.ds((base + t * TILE) * D, TILE * D)], buf, sem)

        def wait_copy(buf, sem):
            pltpu.make_async_copy(
                x_hbm.at[pl.ds(base * D, TILE * D)], buf, sem).wait()

        def compute_tile(t, buf):
            zeros16 = jnp.zeros((16,), jnp.float32)
            for g in range(TILE // 16):
                logit_vecs = [zeros16, zeros16, zeros16]
                for blk in range(16 // TB):
                    tok0 = g * 16 + blk * TB

                    @pl.loop(0, D // 16,
                             init_carry=(zeros16,) * (3 * TB),
                             unroll=D_UNROLL)
                    def accs(c, carry):
                        acc = list(carry)
                        c16 = c * 16
                        w0 = w_v[0, pl.ds(c16, 16)]
                        w1 = w_v[1, pl.ds(c16, 16)]
                        w2 = w_v[2, pl.ds(c16, 16)]
                        for tt in range(TB):
                            xv = buf[pl.ds((tok0 + tt) * D + c16, 16)]
                            acc[3 * tt] = acc[3 * tt] + xv * w0
                            acc[3 * tt + 1] = acc[3 * tt + 1] + xv * w1
                            acc[3 * tt + 2] = acc[3 * tt + 2] + xv * w2
                        return tuple(acc)

                    for tt in range(TB):
                        lane = blk * TB + tt
                        mask = iota == lane
                        for n in range(3):
                            s = jnp.sum(accs[3 * tt + n])
                            logit_vecs[n] = jnp.where(
                                mask, s, logit_vecs[n])

                l0, l1, l2 = logit_vecs
                m = jnp.maximum(jnp.maximum(l0, l1), l2)
                ssum = jnp.exp(l0 - m) + jnp.exp(l1 - m) + jnp.exp(l2 - m)
                off = t * TILE + g * 16
                gt_v[pl.ds(off, 16)] = 1.0 / ssum
                oidx = (off + iota) * N_REC
                plsc.store_scatter(lg_v, [oidx], l0)
                plsc.store_scatter(lg_v, [oidx + 1], l1)
                plsc.store_scatter(lg_v, [oidx + 2], l2)

        start_copy(0, xb0, sem0)
        start_copy(1, xb1, sem1)

        @pl.loop(0, nt, step=2)
        def _(t):
            wait_copy(xb0, sem0)
            compute_tile(t, xb0)

            @pl.when(t + 2 < nt)
            def _():
                start_copy(t + 2, xb0, sem0)

            wait_copy(xb1, sem1)
            compute_tile(t + 1, xb1)

            @pl.when(t + 3 < nt)
            def _():
                start_copy(t + 3, xb1, sem1)

        pltpu.sync_copy(lg_v, lg_hbm.at[pl.ds(base * N_REC, tps * N_REC)])
        pltpu.sync_copy(gt_v, gt_hbm.at[pl.ds(base, tps)])

    return k(xf.reshape(N * D), W)


def kernel(x, W):
    B, T, _ = x.shape
    N = B * T
    xf = x.reshape(N, D)
    lg, gt = _sc_router(xf, W)
    selected = jnp.broadcast_to(
        jnp.arange(T, dtype=jnp.int32)[None, :, None], (B, T, 1))
    return selected, gt.reshape(B, T, 1), lg.reshape(B, T, N_REC)
